# baseline (speedup 1.0000x reference)
import functools

import jax
import jax.numpy as jnp
from jax import lax
from jax.experimental import pallas as pl
from jax.experimental.pallas import tpu as pltpu
from jax.experimental.pallas import tpu_sc as plsc

R = 4
N_NODES = 10000
N_EDGES = 160000
D = 128

NC = 2
NS = 16
NW = NC * NS
B_TOTAL = R * N_EDGES
B_PER_W = B_TOTAL // NW
CHUNK = 400
N_CHUNKS = B_PER_W // CHUNK
NBUF = 2


def _scale_body(attn_ref, msgs_ref, aidx_ref, scaled_ref, idx2_ref):
    r = pl.program_id(0)
    a = attn_ref[r]
    gate = jax.nn.sigmoid(jnp.full((1, D), a, jnp.float32))
    scaled_ref[...] = msgs_ref[...] * gate
    idx2_ref[...] = (aidx_ref[...] + r * N_NODES).reshape(1, 1, N_EDGES)


def _prescale(rel_attn, msgs2d, aidx):
    return pl.pallas_call(
        _scale_body,
        grid=(R,),
        in_specs=[
            pl.BlockSpec(memory_space=pltpu.SMEM),
            pl.BlockSpec((N_NODES, D), lambda r: (r, 0)),
            pl.BlockSpec((N_EDGES,), lambda r: (0,)),
        ],
        out_specs=[
            pl.BlockSpec((N_NODES, D), lambda r: (r, 0)),
            pl.BlockSpec((1, 1, N_EDGES), lambda r: (r, 0, 0)),
        ],
        out_shape=[
            jax.ShapeDtypeStruct((R * N_NODES, D), jnp.float32),
            jax.ShapeDtypeStruct((R, 1, N_EDGES), jnp.int32),
        ],
    )(rel_attn, msgs2d, aidx)


def _sc_gather(table, idx_flat):
    mesh = plsc.VectorSubcoreMesh(core_axis_name="c", subcore_axis_name="s")

    @functools.partial(
        pl.kernel,
        mesh=mesh,
        out_type=jax.ShapeDtypeStruct((B_TOTAL, D), jnp.float32),
        scratch_types=(
            [pltpu.VMEM((B_PER_W,), jnp.int32),
             pltpu.VMEM((NBUF, CHUNK, D), jnp.float32)]
            + [pltpu.SemaphoreType.DMA] * (2 * NBUF)
        ),
    )
    def k(table_hbm, idx_hbm, out_hbm, idx_v, rows_v, *sems):
        gsem = sems[:NBUF]
        ssem = sems[NBUF:]
        wid = lax.axis_index("s") * NC + lax.axis_index("c")
        base = wid * B_PER_W
        pltpu.sync_copy(idx_hbm.at[pl.ds(base, B_PER_W)], idx_v)

        def g_start(c, buf):
            pltpu.make_async_copy(
                table_hbm.at[idx_v.at[pl.ds(c * CHUNK, CHUNK)]],
                rows_v.at[buf], gsem[buf],
            ).start()

        def g_wait(buf):
            pltpu.make_async_copy(
                table_hbm.at[pl.ds(0, CHUNK)], rows_v.at[buf], gsem[buf]
            ).wait()

        def s_start(c, buf):
            pltpu.make_async_copy(
                rows_v.at[buf], out_hbm.at[pl.ds(base + c * CHUNK, CHUNK)],
                ssem[buf],
            ).start()

        def s_wait(buf):
            pltpu.make_async_copy(
                rows_v.at[buf], out_hbm.at[pl.ds(base, CHUNK)], ssem[buf]
            ).wait()

        for b in range(NBUF):
            g_start(b, b)

        @pl.loop(0, N_CHUNKS, step=NBUF)
        def _(c):
            for half in range(2):
                for b in range(NBUF // 2):
                    buf = half * (NBUF // 2) + b
                    g_wait(buf)
                    s_start(c + buf, buf)

                @pl.when(c + NBUF < N_CHUNKS)
                def _():
                    for b in range(NBUF // 2):
                        buf = half * (NBUF // 2) + b
                        s_wait(buf)
                        g_start(c + NBUF + buf, buf)

        for b in range(NBUF):
            s_wait(b)

    return k(table, idx_flat)


def kernel(rel_attn, per_rel_msgs, actor_idx):
    msgs2d = per_rel_msgs.reshape(R * N_NODES, D)
    aidx = actor_idx.astype(jnp.int32)
    scaled, idx2 = _prescale(rel_attn, msgs2d, aidx)
    out_flat = _sc_gather(scaled, idx2.reshape(B_TOTAL))
    return out_flat.reshape(R, N_EDGES, D)


# speedup vs baseline: 1.0143x; 1.0143x over previous
import functools

import jax
import jax.numpy as jnp
from jax import lax
from jax.experimental import pallas as pl
from jax.experimental.pallas import tpu as pltpu
from jax.experimental.pallas import tpu_sc as plsc

R = 4
N_NODES = 10000
N_EDGES = 160000
D = 128

NC = 2
NS = 16
NW = NC * NS
B_TOTAL = R * N_EDGES
B_PER_W = B_TOTAL // NW
CHUNK = 80
N_CHUNKS = B_PER_W // CHUNK
NBUF = 10


def _scale_body(attn_ref, msgs_ref, aidx_ref, scaled_ref, idx2_ref):
    r = pl.program_id(0)
    a = attn_ref[r]
    gate = jax.nn.sigmoid(jnp.full((1, D), a, jnp.float32))
    scaled_ref[...] = msgs_ref[...] * gate
    idx2_ref[...] = (aidx_ref[...] + r * N_NODES).reshape(1, 1, N_EDGES)


def _prescale(rel_attn, msgs2d, aidx):
    return pl.pallas_call(
        _scale_body,
        grid=(R,),
        in_specs=[
            pl.BlockSpec(memory_space=pltpu.SMEM),
            pl.BlockSpec((N_NODES, D), lambda r: (r, 0)),
            pl.BlockSpec((N_EDGES,), lambda r: (0,)),
        ],
        out_specs=[
            pl.BlockSpec((N_NODES, D), lambda r: (r, 0)),
            pl.BlockSpec((1, 1, N_EDGES), lambda r: (r, 0, 0)),
        ],
        out_shape=[
            jax.ShapeDtypeStruct((R * N_NODES, D), jnp.float32),
            jax.ShapeDtypeStruct((R, 1, N_EDGES), jnp.int32),
        ],
    )(rel_attn, msgs2d, aidx)


def _sc_gather(table, idx_flat):
    mesh = plsc.VectorSubcoreMesh(core_axis_name="c", subcore_axis_name="s")

    @functools.partial(
        pl.kernel,
        mesh=mesh,
        out_type=jax.ShapeDtypeStruct((B_TOTAL, D), jnp.float32),
        scratch_types=(
            [pltpu.VMEM((B_PER_W,), jnp.int32),
             pltpu.VMEM((NBUF, CHUNK, D), jnp.float32)]
            + [pltpu.SemaphoreType.DMA] * (2 * NBUF)
        ),
    )
    def k(table_hbm, idx_hbm, out_hbm, idx_v, rows_v, *sems):
        gsem = sems[:NBUF]
        ssem = sems[NBUF:]
        wid = lax.axis_index("s") * NC + lax.axis_index("c")
        base = wid * B_PER_W
        pltpu.sync_copy(idx_hbm.at[pl.ds(base, B_PER_W)], idx_v)

        def g_start(c, buf):
            pltpu.make_async_copy(
                table_hbm.at[idx_v.at[pl.ds(c * CHUNK, CHUNK)]],
                rows_v.at[buf], gsem[buf],
            ).start()

        def g_wait(buf):
            pltpu.make_async_copy(
                table_hbm.at[pl.ds(0, CHUNK)], rows_v.at[buf], gsem[buf]
            ).wait()

        def s_start(c, buf):
            pltpu.make_async_copy(
                rows_v.at[buf], out_hbm.at[pl.ds(base + c * CHUNK, CHUNK)],
                ssem[buf],
            ).start()

        def s_wait(buf):
            pltpu.make_async_copy(
                rows_v.at[buf], out_hbm.at[pl.ds(base, CHUNK)], ssem[buf]
            ).wait()

        for b in range(NBUF):
            g_start(b, b)

        @pl.loop(0, N_CHUNKS, step=NBUF)
        def _(c):
            for half in range(2):
                for b in range(NBUF // 2):
                    buf = half * (NBUF // 2) + b
                    g_wait(buf)
                    s_start(c + buf, buf)

                @pl.when(c + NBUF < N_CHUNKS)
                def _():
                    for b in range(NBUF // 2):
                        buf = half * (NBUF // 2) + b
                        s_wait(buf)
                        g_start(c + NBUF + buf, buf)

        for b in range(NBUF):
            s_wait(b)

    return k(table, idx_flat)


def kernel(rel_attn, per_rel_msgs, actor_idx):
    msgs2d = per_rel_msgs.reshape(R * N_NODES, D)
    aidx = actor_idx.astype(jnp.int32)
    scaled, idx2 = _prescale(rel_attn, msgs2d, aidx)
    out_flat = _sc_gather(scaled, idx2.reshape(B_TOTAL))
    return out_flat.reshape(R, N_EDGES, D)


# speedup vs baseline: 1.5101x; 1.4888x over previous
import functools

import jax
import jax.numpy as jnp
from jax import lax
from jax.experimental import pallas as pl
from jax.experimental.pallas import tpu as pltpu
from jax.experimental.pallas import tpu_sc as plsc

R = 4
N_NODES = 10000
N_EDGES = 160000
D = 128

NC = 2
NS = 16
B_TOTAL = R * N_EDGES
E_PER_S = N_EDGES // NS
LOAD_ROWS = 1000
N_LOADERS = N_NODES // LOAD_ROWS
CHUNK = 40
N_CHUNKS = E_PER_S // CHUNK
NBUF = 5


def _scale_body(attn_ref, msgs_ref, scaled_ref):
    r = pl.program_id(0)
    a = attn_ref[r]
    gate = jax.nn.sigmoid(jnp.full((1, D), a, jnp.float32))
    scaled_ref[...] = msgs_ref[...] * gate


def _prescale(rel_attn, msgs2d):
    return pl.pallas_call(
        _scale_body,
        grid=(R,),
        in_specs=[
            pl.BlockSpec(memory_space=pltpu.SMEM),
            pl.BlockSpec((N_NODES, D), lambda r: (r, 0)),
        ],
        out_specs=pl.BlockSpec((N_NODES, D), lambda r: (r, 0)),
        out_shape=jax.ShapeDtypeStruct((R * N_NODES, D), jnp.float32),
    )(rel_attn, msgs2d)


def _sc_gather(table, aidx):
    mesh = plsc.VectorSubcoreMesh(core_axis_name="c", subcore_axis_name="s")

    @functools.partial(
        pl.kernel,
        mesh=mesh,
        out_type=jax.ShapeDtypeStruct((B_TOTAL, D), jnp.float32),
        scratch_types=(
            [pltpu.VMEM((E_PER_S,), jnp.int32),
             pltpu.VMEM((NBUF, CHUNK, D), jnp.float32),
             pltpu.VMEM_SHARED((N_NODES, D), jnp.float32)]
            + [pltpu.SemaphoreType.DMA] * (2 * NBUF)
        ),
    )
    def k(table_hbm, idx_hbm, out_hbm, idx_v, rows_v, shared, *sems):
        gsem = sems[:NBUF]
        ssem = sems[NBUF:]
        c = lax.axis_index("c")
        s = lax.axis_index("s")
        pltpu.sync_copy(idx_hbm.at[pl.ds(s * E_PER_S, E_PER_S)], idx_v)

        for p in range(2):
            r = NC * c + p
            @pl.when(s < N_LOADERS)
            def _():
                pltpu.sync_copy(
                    table_hbm.at[pl.ds(r * N_NODES + s * LOAD_ROWS, LOAD_ROWS)],
                    shared.at[pl.ds(s * LOAD_ROWS, LOAD_ROWS)],
                )
            plsc.subcore_barrier()

            out_base = r * N_EDGES + s * E_PER_S

            def g_start(cc, buf):
                pltpu.make_async_copy(
                    shared.at[idx_v.at[pl.ds(cc * CHUNK, CHUNK)]],
                    rows_v.at[buf], gsem[buf],
                ).start()

            def g_wait(buf):
                pltpu.make_async_copy(
                    shared.at[pl.ds(0, CHUNK)], rows_v.at[buf], gsem[buf]
                ).wait()

            def s_start(cc, buf):
                pltpu.make_async_copy(
                    rows_v.at[buf],
                    out_hbm.at[pl.ds(out_base + cc * CHUNK, CHUNK)],
                    ssem[buf],
                ).start()

            def s_wait(buf):
                pltpu.make_async_copy(
                    rows_v.at[buf], out_hbm.at[pl.ds(0, CHUNK)], ssem[buf]
                ).wait()

            for b in range(NBUF):
                g_start(b, b)

            @pl.loop(0, N_CHUNKS, step=NBUF)
            def _(cc):
                for b in range(NBUF):
                    g_wait(b)
                    s_start(cc + b, b)

                @pl.when(cc + NBUF < N_CHUNKS)
                def _():
                    for b in range(NBUF):
                        s_wait(b)
                        g_start(cc + NBUF + b, b)

            for b in range(NBUF):
                s_wait(b)
            plsc.subcore_barrier()

    return k(table, aidx)


def kernel(rel_attn, per_rel_msgs, actor_idx):
    msgs2d = per_rel_msgs.reshape(R * N_NODES, D)
    aidx = actor_idx.astype(jnp.int32)
    scaled = _prescale(rel_attn, msgs2d)
    out_flat = _sc_gather(scaled, aidx)
    return out_flat.reshape(R, N_EDGES, D)
